# baseline (speedup 1.0000x reference)
import functools

import jax
import jax.numpy as jnp
from jax import lax
from jax.experimental import pallas as pl
from jax.experimental.pallas import tpu as pltpu
from jax.experimental.pallas import tpu_sc as plsc

NUM_ROWS = 400
EDIM = 32
XW = 16
OW = 78
RCHUNK = 2


def _body(x_hbm, w_hbm, out_hbm, w_v, x_v, out_v, *, rows_per_worker, hist, num_cores):
    wid = lax.axis_index("s") * num_cores + lax.axis_index("c")
    pltpu.sync_copy(w_hbm, w_v)
    row0 = wid * rows_per_worker
    n_chunks = rows_per_worker // RCHUNK
    tokens_per_chunk = RCHUNK * hist

    def do_chunk(ci, _):
        rbase = row0 + ci * RCHUNK
        pltpu.sync_copy(x_hbm.at[pl.ds(rbase, RCHUNK)], x_v)

        @plsc.parallel_loop(0, tokens_per_chunk // 16, unroll=2)
        def do_group(g):
            tok = g * 16 + lax.iota(jnp.int32, 16)
            r = tok // hist
            h = tok - r * hist

            def col(c):
                return jnp.full((16,), c, jnp.int32)

            va = plsc.load_gather(x_v, [r, h, col(0)])
            ia = jnp.clip(va.astype(jnp.int32), 0, NUM_ROWS - 1)
            vo = plsc.load_gather(x_v, [r, h, col(8)])
            io = jnp.clip(vo.astype(jnp.int32), 0, NUM_ROWS - 1)

            for c in range(EDIM):
                ea = plsc.load_gather(w_v, [ia, col(c)])
                plsc.store_scatter(out_v, [r, h, col(c)], ea)
                eo = plsc.load_gather(w_v, [io, col(c)])
                plsc.store_scatter(out_v, [r, h, col(39 + c)], eo)
            for c in range(7):
                sa = plsc.load_gather(x_v, [r, h, col(1 + c)])
                plsc.store_scatter(out_v, [r, h, col(32 + c)], sa)
                so = plsc.load_gather(x_v, [r, h, col(9 + c)])
                plsc.store_scatter(out_v, [r, h, col(71 + c)], so)

        pltpu.sync_copy(out_v, out_hbm.at[pl.ds(rbase, RCHUNK)])
        return 0

    lax.fori_loop(0, n_chunks, do_chunk, 0)


def kernel(x, W):
    B, H, _ = x.shape
    mesh = plsc.VectorSubcoreMesh(core_axis_name="c", subcore_axis_name="s")
    n_workers = mesh.num_cores * mesh.num_subcores
    rows_per_worker = B // n_workers
    assert rows_per_worker * n_workers == B
    assert rows_per_worker % RCHUNK == 0
    assert (RCHUNK * H) % 16 == 0

    body = functools.partial(
        _body,
        rows_per_worker=rows_per_worker,
        hist=H,
        num_cores=mesh.num_cores,
    )
    out = pl.kernel(
        body,
        out_type=jax.ShapeDtypeStruct((B, H, OW), jnp.float32),
        mesh=mesh,
        compiler_params=pltpu.CompilerParams(
            needs_layout_passes=False,
            use_tc_tiling_on_sc=False,
            disable_bounds_checks=True,
        ),
        scratch_types=[
            pltpu.VMEM((NUM_ROWS, EDIM), jnp.float32),
            pltpu.VMEM((RCHUNK, H, XW), jnp.float32),
            pltpu.VMEM((RCHUNK, H, OW), jnp.float32),
        ],
    )(x, W)
    return out
